# baseline (speedup 1.0000x reference)
import functools

import jax
import jax.numpy as jnp
from jax import lax
from jax.experimental import pallas as pl
from jax.experimental.pallas import tpu as pltpu
from jax.experimental.pallas import tpu_sc as plsc

B = 4
R = 4096
C = 2048
N = R * C
K = int(N * 0.8)
M = N - K

SHIFT = 20
NB = 2048
LANES = 16
NC = 2
NS = 16
NW = NC * NS
WPR = NW // B
EPW = N // WPR
CH = 16384
NCH = EPW // CH
VPC = CH // LANES


def _loss_body(p_ref, t_ref, o_ref):
    p = p_ref[...]
    t = t_ref[...]
    o_ref[...] = (jnp.maximum(p, 0.0) - p * t
                  + jnp.log(1.0 + jnp.exp(-jnp.abs(p))))


_BR = 512
NP = 2
BC = B // NP


def _make_loss_call(off):
    return pl.pallas_call(
        _loss_body,
        grid=(BC, R // _BR),
        in_specs=[
            pl.BlockSpec((1, _BR, C), lambda b, r, off=off: (b + off, r, 0)),
            pl.BlockSpec((1, _BR, C), lambda b, r, off=off: (b + off, r, 0)),
        ],
        out_specs=pl.BlockSpec((1, _BR, C), lambda b, r: (b, r, 0)),
        out_shape=jax.ShapeDtypeStruct((BC, R, C), jnp.float32),
    )


_loss_calls = [_make_loss_call(ci * BC) for ci in range(NP)]


_mesh = plsc.VectorSubcoreMesh(
    core_axis_name="c", subcore_axis_name="s",
    num_cores=NC, num_subcores=NS)


def _make_sc_hist(nrows):
    epw = nrows * N // NW
    nch = epw // CH

    @functools.partial(
        pl.kernel,
        out_type=(
            jax.ShapeDtypeStruct((NW, LANES, NB), jnp.float32),
            jax.ShapeDtypeStruct((NW, LANES, NB), jnp.float32),
        ),
        mesh=_mesh,
        scratch_types=[
            pltpu.VMEM((CH,), jnp.float32),
            pltpu.VMEM((CH,), jnp.float32),
            pltpu.VMEM((LANES, NB), jnp.float32),
            pltpu.VMEM((LANES, NB), jnp.float32),
            pltpu.SemaphoreType.DMA,
            pltpu.SemaphoreType.DMA,
        ],
        compiler_params=pltpu.CompilerParams(
            needs_layout_passes=False, use_tc_tiling_on_sc=False),
    )
    def _sc_hist(loss_hbm, cnt_hbm, sum_hbm,
                 buf0, buf1, cnt_v, sum_v, sem0, sem1):
        wid = lax.axis_index("c") * NS + lax.axis_index("s")
        base = wid * epw

        zeros16 = jnp.zeros((LANES,), jnp.float32)

        def _zinit(i, carry):
            for r in range(LANES):
                cnt_v[r, pl.ds(i * LANES, LANES)] = zeros16
                sum_v[r, pl.ds(i * LANES, LANES)] = zeros16
            return carry

        lax.fori_loop(0, NB // LANES, _zinit, 0)

        lane = lax.iota(jnp.int32, LANES)
        ones16 = jnp.ones((LANES,), jnp.float32)

        def _start(buf, sem, ci):
            pltpu.async_copy(loss_hbm.at[pl.ds(base + ci * CH, CH)], buf, sem)

        def _wait(buf, sem, ci):
            pltpu.make_async_copy(
                loss_hbm.at[pl.ds(base + ci * CH, CH)], buf, sem).wait()

        U = 8

        def _proc(buf):
            @plsc.parallel_loop(0, VPC, 1, unroll=U)
            def _vec(vi):
                v = buf[pl.ds(vi * LANES, LANES)]
                bits = lax.bitcast_convert_type(v, jnp.int32)
                bin_ = lax.shift_right_logical(bits, SHIFT)
                plsc.addupdate_scatter(cnt_v, [lane, bin_], ones16)
                plsc.addupdate_scatter(sum_v, [lane, bin_], v)

        _start(buf0, sem0, 0)

        def _pair(pj, carry):
            c0 = pj * 2
            _wait(buf0, sem0, c0)
            _start(buf1, sem1, c0 + 1)
            _proc(buf0)
            _wait(buf1, sem1, c0 + 1)

            @pl.when(c0 + 2 < nch)
            def _():
                _start(buf0, sem0, c0 + 2)

            _proc(buf1)
            return carry

        lax.fori_loop(0, nch // 2, _pair, 0)

        pltpu.sync_copy(cnt_v, cnt_hbm.at[wid])
        pltpu.sync_copy(sum_v, sum_hbm.at[wid])

    return _sc_hist


_sc_hist_chunk = _make_sc_hist(BC)


def _select_body(*refs):
    cnt_refs = refs[:NP]
    sum_refs = refs[NP:2 * NP]
    o_ref = refs[2 * NP]
    hpr = (NW // BC) * LANES
    c4 = jnp.concatenate(
        [r[...].reshape(BC, hpr, NB).sum(axis=1) for r in cnt_refs], axis=0)
    s4 = jnp.concatenate(
        [r[...].reshape(BC, hpr, NB).sum(axis=1) for r in sum_refs], axis=0)

    jj = lax.broadcasted_iota(jnp.int32, (NB, NB), 0)
    bb = lax.broadcasted_iota(jnp.int32, (NB, NB), 1)
    lt = (jj <= bb).astype(jnp.float32)
    pc = jax.lax.dot(c4, lt, precision=jax.lax.Precision.HIGHEST)
    ps = jax.lax.dot(s4, lt, precision=jax.lax.Precision.HIGHEST)
    total_s = ps[:, NB - 1]

    m = jnp.float32(M)
    cb = pc - c4
    star = jnp.logical_and(cb <= m, pc >= m + 1.0).astype(jnp.float32)

    bidx = lax.broadcasted_iota(jnp.int32, (B, NB), 1)
    lo = lax.bitcast_convert_type(bidx << SHIFT, jnp.float32)
    hi = lax.bitcast_convert_type((bidx + 1) << SHIFT, jnp.float32)
    w = jnp.maximum(hi - lo, 0.0)

    c_star = jnp.sum(star * c4, axis=1)
    cb_star = jnp.sum(star * cb, axis=1)
    s_star = jnp.sum(star * s4, axis=1)
    ps_star = jnp.sum(star * ps, axis=1)
    w_star = jnp.sum(star * w, axis=1)

    kept = c_star - (m - cb_star)
    sum_above = total_s - ps_star
    avg = s_star / c_star
    partial = kept * (avg + w_star * (c_star - kept) / (2.0 * c_star))
    result = jnp.sum(sum_above + partial) / jnp.float32(B * K)
    o_ref[...] = result.reshape(1, 1)


_select_call = pl.pallas_call(
    _select_body,
    out_shape=jax.ShapeDtypeStruct((1, 1), jnp.float32),
)


def kernel(pred, true):
    cnts, sums = [], []
    for ci in range(NP):
        loss = _loss_calls[ci](pred, true)
        cnt, ssum = _sc_hist_chunk(loss.reshape(BC * N))
        cnts.append(cnt.reshape(NW * LANES, NB))
        sums.append(ssum.reshape(NW * LANES, NB))
    out = _select_call(*cnts, *sums)
    return out.reshape(())


# speedup vs baseline: 1.2368x; 1.2368x over previous
import functools

import jax
import jax.numpy as jnp
from jax import lax
from jax.experimental import pallas as pl
from jax.experimental.pallas import tpu as pltpu
from jax.experimental.pallas import tpu_sc as plsc

B = 4
R = 4096
C = 2048
N = R * C
K = int(N * 0.8)
M = N - K

SHIFT = 20
NB = 2048
LANES = 16
NC = 2
NS = 16
NW = NC * NS
WPR = NW // B
EPW = N // WPR
CH = 16384
NCH = EPW // CH
VPC = CH // LANES


def _loss_body(p_ref, t_ref, o_ref):
    p = p_ref[...]
    t = t_ref[...]
    l = (jnp.maximum(p, 0.0) - p * t
         + jnp.log(1.0 + jnp.exp(-jnp.abs(p))))
    r = lax.bitcast_convert_type(l, jnp.int32) + jnp.int32(0x8000)
    hi = r[:, :, : C // 2] & jnp.int32(-65536)
    lo = lax.shift_right_logical(r[:, :, C // 2:], 16)
    o_ref[...] = hi | lo


_BR = 512
NP = 1
BC = B // NP
CW = C // 2


def _make_loss_call(off):
    return pl.pallas_call(
        _loss_body,
        grid=(BC, R // _BR),
        in_specs=[
            pl.BlockSpec((1, _BR, C), lambda b, r, off=off: (b + off, r, 0)),
            pl.BlockSpec((1, _BR, C), lambda b, r, off=off: (b + off, r, 0)),
        ],
        out_specs=pl.BlockSpec((1, _BR, CW), lambda b, r: (b, r, 0)),
        out_shape=jax.ShapeDtypeStruct((BC, R, CW), jnp.int32),
    )


_loss_calls = [_make_loss_call(ci * BC) for ci in range(NP)]


_mesh = plsc.VectorSubcoreMesh(
    core_axis_name="c", subcore_axis_name="s",
    num_cores=NC, num_subcores=NS)


def _make_sc_hist(nrows):
    epw = nrows * (N // 2) // NW
    nch = epw // CH

    @functools.partial(
        pl.kernel,
        out_type=(
            jax.ShapeDtypeStruct((NW, LANES, NB), jnp.float32),
            jax.ShapeDtypeStruct((NW, LANES, NB), jnp.float32),
        ),
        mesh=_mesh,
        scratch_types=[
            pltpu.VMEM((CH,), jnp.int32),
            pltpu.VMEM((CH,), jnp.int32),
            pltpu.VMEM((LANES, NB), jnp.float32),
            pltpu.VMEM((LANES, NB), jnp.float32),
            pltpu.SemaphoreType.DMA,
            pltpu.SemaphoreType.DMA,
        ],
        compiler_params=pltpu.CompilerParams(
            needs_layout_passes=False, use_tc_tiling_on_sc=False),
    )
    def _sc_hist(loss_hbm, cnt_hbm, sum_hbm,
                 buf0, buf1, cnt_v, sum_v, sem0, sem1):
        wid = lax.axis_index("c") * NS + lax.axis_index("s")
        base = wid * epw

        zeros16 = jnp.zeros((LANES,), jnp.float32)

        def _zinit(i, carry):
            for r in range(LANES):
                cnt_v[r, pl.ds(i * LANES, LANES)] = zeros16
                sum_v[r, pl.ds(i * LANES, LANES)] = zeros16
            return carry

        lax.fori_loop(0, NB // LANES, _zinit, 0)

        lane = lax.iota(jnp.int32, LANES)
        ones16 = jnp.ones((LANES,), jnp.float32)

        def _start(buf, sem, ci):
            pltpu.async_copy(loss_hbm.at[pl.ds(base + ci * CH, CH)], buf, sem)

        def _wait(buf, sem, ci):
            pltpu.make_async_copy(
                loss_hbm.at[pl.ds(base + ci * CH, CH)], buf, sem).wait()

        U = 8

        def _proc(buf):
            @plsc.parallel_loop(0, VPC, 1, unroll=U)
            def _vec(vi):
                v = buf[pl.ds(vi * LANES, LANES)]
                vhi = lax.bitcast_convert_type(v & jnp.int32(-65536),
                                               jnp.float32)
                vlo = lax.bitcast_convert_type(
                    lax.shift_left(v, 16), jnp.float32)
                bin_hi = lax.shift_right_logical(v, SHIFT)
                bin_lo = (lax.shift_right_logical(v, SHIFT - 16)
                          & jnp.int32(NB - 1))
                plsc.addupdate_scatter(cnt_v, [lane, bin_hi], ones16)
                plsc.addupdate_scatter(sum_v, [lane, bin_hi], vhi)
                plsc.addupdate_scatter(cnt_v, [lane, bin_lo], ones16)
                plsc.addupdate_scatter(sum_v, [lane, bin_lo], vlo)

        _start(buf0, sem0, 0)

        def _pair(pj, carry):
            c0 = pj * 2
            _wait(buf0, sem0, c0)
            _start(buf1, sem1, c0 + 1)
            _proc(buf0)
            _wait(buf1, sem1, c0 + 1)

            @pl.when(c0 + 2 < nch)
            def _():
                _start(buf0, sem0, c0 + 2)

            _proc(buf1)
            return carry

        lax.fori_loop(0, nch // 2, _pair, 0)

        pltpu.sync_copy(cnt_v, cnt_hbm.at[wid])
        pltpu.sync_copy(sum_v, sum_hbm.at[wid])

    return _sc_hist


_sc_hist_chunk = _make_sc_hist(BC)


def _select_body(*refs):
    cnt_refs = refs[:NP]
    sum_refs = refs[NP:2 * NP]
    o_ref = refs[2 * NP]
    hpr = (NW // BC) * LANES
    c4 = jnp.concatenate(
        [r[...].reshape(BC, hpr, NB).sum(axis=1) for r in cnt_refs], axis=0)
    s4 = jnp.concatenate(
        [r[...].reshape(BC, hpr, NB).sum(axis=1) for r in sum_refs], axis=0)

    jj = lax.broadcasted_iota(jnp.int32, (NB, NB), 0)
    bb = lax.broadcasted_iota(jnp.int32, (NB, NB), 1)
    lt = (jj <= bb).astype(jnp.float32)
    pc = jax.lax.dot(c4, lt, precision=jax.lax.Precision.HIGHEST)
    ps = jax.lax.dot(s4, lt, precision=jax.lax.Precision.HIGHEST)
    total_s = ps[:, NB - 1]

    m = jnp.float32(M)
    cb = pc - c4
    star = jnp.logical_and(cb <= m, pc >= m + 1.0).astype(jnp.float32)

    bidx = lax.broadcasted_iota(jnp.int32, (B, NB), 1)
    lo = lax.bitcast_convert_type(bidx << SHIFT, jnp.float32)
    hi = lax.bitcast_convert_type((bidx + 1) << SHIFT, jnp.float32)
    w = jnp.maximum(hi - lo, 0.0)

    c_star = jnp.sum(star * c4, axis=1)
    cb_star = jnp.sum(star * cb, axis=1)
    s_star = jnp.sum(star * s4, axis=1)
    ps_star = jnp.sum(star * ps, axis=1)
    w_star = jnp.sum(star * w, axis=1)

    kept = c_star - (m - cb_star)
    sum_above = total_s - ps_star
    avg = s_star / c_star
    partial = kept * (avg + w_star * (c_star - kept) / (2.0 * c_star))
    result = jnp.sum(sum_above + partial) / jnp.float32(B * K)
    o_ref[...] = result.reshape(1, 1)


_select_call = pl.pallas_call(
    _select_body,
    out_shape=jax.ShapeDtypeStruct((1, 1), jnp.float32),
)


def kernel(pred, true):
    cnts, sums = [], []
    for ci in range(NP):
        loss = _loss_calls[ci](pred, true)
        cnt, ssum = _sc_hist_chunk(loss.reshape(BC * N // 2))
        cnts.append(cnt.reshape(NW * LANES, NB))
        sums.append(ssum.reshape(NW * LANES, NB))
    out = _select_call(*cnts, *sums)
    return out.reshape(())
